# Initial kernel scaffold; baseline (speedup 1.0000x reference)
import jax
import jax.numpy as jnp
from jax.experimental import pallas as pl


def kernel(src, W):
    raise NotImplementedError("write your pallas kernel here")


# speedup vs baseline: 1.1870x; 1.1870x over previous
import functools

import jax
import jax.numpy as jnp
from jax import lax
from jax.experimental import pallas as pl
from jax.experimental.pallas import tpu as pltpu
from jax.experimental.pallas import tpu_sc as plsc

_info = plsc.get_sparse_core_info()
_NC, _NS = _info.num_cores, _info.num_subcores
_NW = _NC * _NS

_C = 128


def _gather_rows(table, idx3):
    nw, nchunk, c = idx3.shape
    _, d = table.shape
    mesh = plsc.VectorSubcoreMesh(core_axis_name="c", subcore_axis_name="s")

    @functools.partial(
        pl.kernel,
        out_type=jax.ShapeDtypeStruct((nw, nchunk, c, d), jnp.float32),
        mesh=mesh,
        scratch_types=[
            pltpu.VMEM((nchunk, c), jnp.int32),
            pltpu.VMEM((c, d), jnp.float32),
            pltpu.SemaphoreType.DMA,
        ],
        compiler_params=pltpu.CompilerParams(use_tc_tiling_on_sc=False),
    )
    def k(table_hbm, idx_hbm, out_hbm, idx_v, rows_v, sem):
        wid = lax.axis_index("s") * _NC + lax.axis_index("c")
        pltpu.sync_copy(idx_hbm.at[wid], idx_v)

        def body(j, _):
            pltpu.async_copy(table_hbm.at[idx_v.at[j]], rows_v, sem).wait()
            pltpu.sync_copy(rows_v, out_hbm.at[wid, j])
            return _

        lax.fori_loop(0, nchunk, body, None)

    return k(table, idx3)


def kernel(src, W):
    b, h = src.shape
    _, d = W.shape
    n = b * h
    assert n % (_NW * _C) == 0
    nchunk = n // (_NW * _C)
    idx3 = src.reshape(_NW, nchunk, _C)
    out = _gather_rows(W, idx3)
    return out.reshape(b, h, d)


# speedup vs baseline: 1.2414x; 1.0458x over previous
import functools

import jax
import jax.numpy as jnp
from jax import lax
from jax.experimental import pallas as pl
from jax.experimental.pallas import tpu as pltpu
from jax.experimental.pallas import tpu_sc as plsc

_info = plsc.get_sparse_core_info()
_NC, _NS = _info.num_cores, _info.num_subcores
_NW = _NC * _NS

_C = 1024


def _gather_rows(table, idx3):
    nw, nchunk, c = idx3.shape
    _, d = table.shape
    mesh = plsc.VectorSubcoreMesh(core_axis_name="c", subcore_axis_name="s")

    @functools.partial(
        pl.kernel,
        out_type=jax.ShapeDtypeStruct((nw, nchunk, c, d), jnp.float32),
        mesh=mesh,
        scratch_types=[
            pltpu.VMEM((nchunk, c), jnp.int32),
            pltpu.VMEM((2, c, d), jnp.float32),
            pltpu.SemaphoreType.DMA,
            pltpu.SemaphoreType.DMA,
            pltpu.SemaphoreType.DMA,
            pltpu.SemaphoreType.DMA,
        ],
        compiler_params=pltpu.CompilerParams(use_tc_tiling_on_sc=False),
    )
    def k(table_hbm, idx_hbm, out_hbm, idx_v, rows_v, sg0, sg1, sw0, sw1):
        wid = lax.axis_index("s") * _NC + lax.axis_index("c")
        pltpu.sync_copy(idx_hbm.at[wid], idx_v)

        sg, sw = [sg0, sg1], [sw0, sw1]
        gd, wd = [None] * nchunk, [None] * nchunk

        def g_start(i):
            b = i & 1
            gd[i] = pltpu.async_copy(table_hbm.at[idx_v.at[i]], rows_v.at[b], sg[b])

        def w_start(i):
            b = i & 1
            wd[i] = pltpu.async_copy(rows_v.at[b], out_hbm.at[wid, i], sw[b])

        g_start(0)
        for i in range(nchunk):
            if i + 1 < nchunk:
                if i >= 1:
                    wd[i - 1].wait()
                g_start(i + 1)
            gd[i].wait()
            w_start(i)
        wd[nchunk - 2].wait()
        wd[nchunk - 1].wait()

    return k(table, idx3)


def kernel(src, W):
    b, h = src.shape
    _, d = W.shape
    n = b * h
    assert n % (_NW * _C) == 0
    nchunk = n // (_NW * _C)
    idx3 = src.reshape(_NW, nchunk, _C)
    out = _gather_rows(W, idx3)
    return out.reshape(b, h, d)


# speedup vs baseline: 2.0250x; 1.6312x over previous
import functools

import jax
import jax.numpy as jnp
from jax import lax
from jax.experimental import pallas as pl
from jax.experimental.pallas import tpu as pltpu
from jax.experimental.pallas import tpu_sc as plsc

_info = plsc.get_sparse_core_info()
_NC, _NS = _info.num_cores, _info.num_subcores
_NW = _NC * _NS
_L = 16
_PK = 4


def _gather_t(table4, src_t, d):
    v4, dp = table4.shape
    h, b = src_t.shape
    blk = b // _NW
    half = blk // 2
    ng = half // _L
    mesh = plsc.VectorSubcoreMesh(core_axis_name="c", subcore_axis_name="s")

    @functools.partial(
        pl.kernel,
        out_type=jax.ShapeDtypeStruct((h, d, b), jnp.float32),
        mesh=mesh,
        scratch_types=[
            pltpu.VMEM((h, blk), jnp.int32),
            pltpu.VMEM((half,), jnp.int32),
            pltpu.VMEM((half,), jnp.int32),
            pltpu.VMEM((half, dp), jnp.float32),
            pltpu.VMEM((half, dp), jnp.float32),
            pltpu.VMEM((d, blk), jnp.float32),
            pltpu.SemaphoreType.DMA,
            pltpu.SemaphoreType.DMA,
            pltpu.SemaphoreType.DMA,
        ],
        compiler_params=pltpu.CompilerParams(
            use_tc_tiling_on_sc=True, needs_layout_passes=False
        ),
    )
    def k(table_hbm, src_hbm, out_hbm, idx_v, q0, q1, rows0, rows1, rt,
          sg0, sg1, sw):
        wid = lax.axis_index("s") * _NC + lax.axis_index("c")
        b0 = wid * blk
        pltpu.sync_copy(src_hbm.at[:, pl.ds(b0, blk)], idx_v)

        rows, q, sg = [rows0, rows1], [q0, q1], [sg0, sg1]
        lane = lax.iota(jnp.int32, _L)

        def prep(hh, c):
            @plsc.parallel_loop(0, ng, unroll=4)
            def pbody(g, c=c):
                iv = idx_v[hh, pl.ds(c * half + g * _L, _L)]
                q[c][pl.ds(g * _L, _L)] = lax.shift_right_logical(iv, 2)

        def g_start(c):
            pltpu.async_copy(table_hbm.at[q[c]], rows[c], sg[c])

        def g_wait(c):
            pltpu.make_async_copy(table_hbm.at[pl.ds(0, half)], rows[c], sg[c]).wait()

        def w_wait():
            pltpu.make_async_copy(rt, out_hbm.at[0, :, pl.ds(b0, blk)], sw).wait()

        def tpose(hh, c):
            @plsc.parallel_loop(0, ng, unroll=2)
            def tbody(g, c=c):
                iv = idx_v[hh, pl.ds(c * half + g * _L, _L)]
                colbase = lax.mul(lax.band(iv, 3), d) if False else (iv & 3) * d
                rvec = g * _L + lane
                for e in range(d):
                    vec = plsc.load_gather(rows[c], [rvec, colbase + e])
                    rt[e, pl.ds(c * half + g * _L, _L)] = vec

        prep(0, 0)
        prep(0, 1)
        g_start(0)
        g_start(1)

        def body(hh, _):
            g_wait(0)

            @pl.when(hh >= 1)
            def _():
                w_wait()

            tpose(hh, 0)
            g_wait(1)

            @pl.when(hh + 1 < h)
            def _():
                prep(hh + 1, 0)
                g_start(0)

            tpose(hh, 1)

            @pl.when(hh + 1 < h)
            def _():
                prep(hh + 1, 1)
                g_start(1)

            pltpu.async_copy(rt, out_hbm.at[hh, :, pl.ds(b0, blk)], sw)
            return _

        lax.fori_loop(0, h, body, None)
        w_wait()

    return k(table4, src_t)


def kernel(src, W):
    b, h = src.shape
    v, d = W.shape
    assert v % _PK == 0 and b % (2 * _NW) == 0
    w4 = W.reshape(v // _PK, _PK * d)
    out_t = _gather_t(w4, src.T, d)
    return out_t.transpose(2, 0, 1)
